# baseline (speedup 1.0000x reference)
import jax
import jax.numpy as jnp
from jax.experimental import pallas as pl

B = 1024
D2 = 4096
F = 16384
K = 64


_BN_ENC = 512


def _encode_body(x_ref, w_ref, b_ref, o_ref):
    w = w_ref[...].reshape(D2, _BN_ENC)
    acc = jnp.dot(x_ref[...], w, preferred_element_type=jnp.float32)
    o_ref[...] = jnp.maximum(acc + b_ref[...], 0.0)


def _encode(xf, We, be):
    grid = (F // _BN_ENC,)
    return pl.pallas_call(
        _encode_body,
        grid=grid,
        in_specs=[
            pl.BlockSpec((B, D2), lambda n: (0, 0)),
            pl.BlockSpec((2, D2 // 2, _BN_ENC), lambda n: (0, 0, n)),
            pl.BlockSpec((1, _BN_ENC), lambda n: (0, n)),
        ],
        out_specs=pl.BlockSpec((B, _BN_ENC), lambda n: (0, n)),
        out_shape=jax.ShapeDtypeStruct((B, F), jnp.float32),
    )(xf, We, be)


_BM_TOP = 128


def _topk_body(pre_ref, o_ref):
    pre = pre_ref[...]
    bits = jax.lax.bitcast_convert_type(pre, jnp.int32)
    lo = jnp.zeros((_BM_TOP, 1), jnp.int32)
    hi = jnp.full((_BM_TOP, 1), 0x7F800000, jnp.int32)

    def step(_, carry):
        lo, hi = carry
        mid = lo + ((hi - lo) >> 1)
        cnt = jnp.sum((bits >= mid).astype(jnp.int32), axis=1, keepdims=True)
        ge = cnt >= K
        return jnp.where(ge, mid, lo), jnp.where(ge, hi, mid)

    lo, hi = jax.lax.fori_loop(0, 31, step, (lo, hi))
    o_ref[...] = jnp.where(bits >= lo, pre, 0.0)


def _topk_mask(pre):
    grid = (B // _BM_TOP,)
    return pl.pallas_call(
        _topk_body,
        grid=grid,
        in_specs=[pl.BlockSpec((_BM_TOP, F), lambda m: (m, 0))],
        out_specs=pl.BlockSpec((_BM_TOP, F), lambda m: (m, 0)),
        out_shape=jax.ShapeDtypeStruct((B, F), jnp.float32),
    )(pre)


_BK_DEC = 512


def _decode_body(f_ref, w_ref, b_ref, o_ref):
    k = pl.program_id(0)

    @pl.when(k == 0)
    def _():
        o_ref[...] = jnp.broadcast_to(b_ref[...], o_ref.shape)

    fb = f_ref[...].astype(jnp.bfloat16)
    o_ref[...] += jnp.dot(fb, w_ref[...], preferred_element_type=jnp.float32)


def _decode(feat, Wd, bd):
    grid = (F // _BK_DEC,)
    return pl.pallas_call(
        _decode_body,
        grid=grid,
        in_specs=[
            pl.BlockSpec((B, _BK_DEC), lambda k: (0, k)),
            pl.BlockSpec((_BK_DEC, D2), lambda k: (k, 0)),
            pl.BlockSpec((1, D2), lambda k: (0, 0)),
        ],
        out_specs=pl.BlockSpec((B, D2), lambda k: (0, 0)),
        out_shape=jax.ShapeDtypeStruct((B, D2), jnp.float32),
    )(feat, Wd, bd)


def kernel(x, W_enc, b_enc, W_dec, b_dec):
    xf = x.reshape(B, D2).astype(jnp.bfloat16)
    be = b_enc.reshape(1, F)
    Wd = W_dec.reshape(F, D2).astype(jnp.bfloat16)
    bd = b_dec.reshape(1, D2)

    pre = _encode(xf, W_enc.astype(jnp.bfloat16), be)
    features = _topk_mask(pre)
    recon = _decode(features, Wd, bd).reshape(B, 2, D2 // 2)
    return recon, features


# speedup vs baseline: 1.2463x; 1.2463x over previous
import jax
import jax.numpy as jnp
from jax.experimental import pallas as pl

B = 1024
D2 = 4096
F = 16384
K = 64


_BN_ENC = 1024


def _encode_body(x_ref, w_ref, b_ref, o_ref):
    w = w_ref[...].reshape(D2, _BN_ENC)
    acc = jnp.dot(x_ref[...], w, preferred_element_type=jnp.float32)
    o_ref[...] = jnp.maximum(acc + b_ref[...], 0.0)


def _encode(xf, We, be):
    grid = (F // _BN_ENC,)
    return pl.pallas_call(
        _encode_body,
        grid=grid,
        in_specs=[
            pl.BlockSpec((B, D2), lambda n: (0, 0)),
            pl.BlockSpec((2, D2 // 2, _BN_ENC), lambda n: (0, 0, n)),
            pl.BlockSpec((1, _BN_ENC), lambda n: (0, n)),
        ],
        out_specs=pl.BlockSpec((B, _BN_ENC), lambda n: (0, n)),
        out_shape=jax.ShapeDtypeStruct((B, F), jnp.float32),
    )(xf, We, be)


_BM_TOP = 128


def _topk_body(pre_ref, o_ref):
    pre = pre_ref[...]
    bits = jax.lax.bitcast_convert_type(pre, jnp.int32)
    lo = jnp.zeros((_BM_TOP, 1), jnp.int32)
    hi = jnp.full((_BM_TOP, 1), 0x7F800000, jnp.int32)

    def step(_, carry):
        lo, hi = carry
        mid = lo + ((hi - lo) >> 1)
        cnt = jnp.sum((bits >= mid).astype(jnp.int32), axis=1, keepdims=True)
        ge = cnt >= K
        return jnp.where(ge, mid, lo), jnp.where(ge, hi, mid)

    lo, hi = jax.lax.fori_loop(0, 31, step, (lo, hi))
    o_ref[...] = jnp.where(bits >= lo, pre, 0.0)


def _topk_mask(pre):
    grid = (B // _BM_TOP,)
    return pl.pallas_call(
        _topk_body,
        grid=grid,
        in_specs=[pl.BlockSpec((_BM_TOP, F), lambda m: (m, 0))],
        out_specs=pl.BlockSpec((_BM_TOP, F), lambda m: (m, 0)),
        out_shape=jax.ShapeDtypeStruct((B, F), jnp.float32),
    )(pre)


_BK_DEC = 512


def _decode_body(f_ref, w_ref, b_ref, o_ref):
    k = pl.program_id(0)

    @pl.when(k == 0)
    def _():
        o_ref[...] = jnp.broadcast_to(b_ref[...], o_ref.shape)

    fb = f_ref[...].astype(jnp.bfloat16)
    wb = w_ref[...].astype(jnp.bfloat16)
    o_ref[...] += jnp.dot(fb, wb, preferred_element_type=jnp.float32)


def _decode(feat, Wd, bd):
    grid = (F // _BK_DEC,)
    return pl.pallas_call(
        _decode_body,
        grid=grid,
        in_specs=[
            pl.BlockSpec((B, _BK_DEC), lambda k: (0, k)),
            pl.BlockSpec((_BK_DEC, D2), lambda k: (k, 0)),
            pl.BlockSpec((1, D2), lambda k: (0, 0)),
        ],
        out_specs=pl.BlockSpec((B, D2), lambda k: (0, 0)),
        out_shape=jax.ShapeDtypeStruct((B, D2), jnp.float32),
    )(feat, Wd, bd)


def kernel(x, W_enc, b_enc, W_dec, b_dec):
    xf = x.reshape(B, D2)
    be = b_enc.reshape(1, F)
    Wd = W_dec.reshape(F, D2)
    bd = b_dec.reshape(1, D2)

    pre = _encode(xf, W_enc, be)
    features = _topk_mask(pre)
    recon = _decode(features, Wd, bd).reshape(B, 2, D2 // 2)
    return recon, features


# speedup vs baseline: 1.3665x; 1.0965x over previous
import jax
import jax.numpy as jnp
from jax.experimental import pallas as pl
from jax.experimental.pallas import tpu as pltpu

B = 1024
D2 = 4096
F = 16384
K = 64


_BN_ENC = 512


def _encode_body(x_ref, w_ref, b_ref, o_ref):
    w = w_ref[...].reshape(D2, _BN_ENC)
    acc = jnp.dot(x_ref[...], w, preferred_element_type=jnp.float32)
    o_ref[...] = jnp.maximum(acc + b_ref[...], 0.0)


def _encode(xf, We, be):
    grid = (F // _BN_ENC,)
    return pl.pallas_call(
        _encode_body,
        grid=grid,
        in_specs=[
            pl.BlockSpec((B, D2), lambda n: (0, 0)),
            pl.BlockSpec((2, D2 // 2, _BN_ENC), lambda n: (0, 0, n)),
            pl.BlockSpec((1, _BN_ENC), lambda n: (0, n)),
        ],
        out_specs=pl.BlockSpec((B, _BN_ENC), lambda n: (0, n)),
        out_shape=jax.ShapeDtypeStruct((B, F), jnp.float32),
    )(xf, We, be)


_BM_TOP = 128


def _topk_body(pre_ref, o_ref):
    pre = pre_ref[...]
    bits = jax.lax.bitcast_convert_type(pre, jnp.int32)
    hi16 = (bits >> 16).astype(jnp.int16)
    lo16 = (bits & 0xFFFF).astype(jnp.int16)
    smin = jnp.int16(-0x8000)
    lo16 = lo16 ^ smin
    one16 = jnp.ones(hi16.shape, jnp.int16)
    zero16 = jnp.zeros(hi16.shape, jnp.int16)

    def count_ge(mask16):
        ones = jnp.where(mask16, one16, zero16)
        packed = pltpu.bitcast(ones, jnp.int32)
        s = jnp.sum(packed, axis=1, keepdims=True)
        return s & 0xFFFF, s >> 16

    def bcast16(vlo, vhi):
        word = (vlo & 0xFFFF) | (vhi << 16)
        return pltpu.bitcast(word, jnp.int16)

    kk = jnp.int32(K)
    t1a = jnp.zeros((_BM_TOP // 2, 1), jnp.int32)
    t1b = jnp.zeros((_BM_TOP // 2, 1), jnp.int32)
    for j in range(14, -1, -1):
        ca, cb = t1a | (1 << j), t1b | (1 << j)
        cnt_a, cnt_b = count_ge(hi16 >= bcast16(ca, cb))
        t1a = jnp.where(cnt_a >= kk, ca, t1a)
        t1b = jnp.where(cnt_b >= kk, cb, t1b)

    p16 = bcast16(t1a, t1b)
    band = hi16 == p16
    ch_a, ch_b = count_ge(hi16 > p16)

    t2a = jnp.zeros((_BM_TOP // 2, 1), jnp.int32)
    t2b = jnp.zeros((_BM_TOP // 2, 1), jnp.int32)
    for j in range(15, -1, -1):
        ca, cb = t2a | (1 << j), t2b | (1 << j)
        cand16 = bcast16(ca ^ 0x8000, cb ^ 0x8000)
        cnt_a, cnt_b = count_ge(band & (lo16 >= cand16))
        t2a = jnp.where(ch_a + cnt_a >= kk, ca, t2a)
        t2b = jnp.where(ch_b + cnt_b >= kk, cb, t2b)

    t2_16 = bcast16(t2a ^ 0x8000, t2b ^ 0x8000)
    keep = (hi16 > p16) | (band & (lo16 >= t2_16))
    o_ref[...] = jnp.where(keep, pre, 0.0)


def _topk_mask(pre):
    grid = (B // _BM_TOP,)
    return pl.pallas_call(
        _topk_body,
        grid=grid,
        in_specs=[pl.BlockSpec((_BM_TOP, F), lambda m: (m, 0))],
        out_specs=pl.BlockSpec((_BM_TOP, F), lambda m: (m, 0)),
        out_shape=jax.ShapeDtypeStruct((B, F), jnp.float32),
    )(pre)


_BK_DEC = 512


def _decode_body(f_ref, w_ref, b_ref, o_ref):
    k = pl.program_id(0)

    @pl.when(k == 0)
    def _():
        o_ref[...] = jnp.broadcast_to(b_ref[...], o_ref.shape)

    fb = f_ref[...].astype(jnp.bfloat16)
    wb = w_ref[...].astype(jnp.bfloat16)
    o_ref[...] += jnp.dot(fb, wb, preferred_element_type=jnp.float32)


def _decode(feat, Wd, bd):
    grid = (F // _BK_DEC,)
    return pl.pallas_call(
        _decode_body,
        grid=grid,
        in_specs=[
            pl.BlockSpec((B, _BK_DEC), lambda k: (0, k)),
            pl.BlockSpec((_BK_DEC, D2), lambda k: (k, 0)),
            pl.BlockSpec((1, D2), lambda k: (0, 0)),
        ],
        out_specs=pl.BlockSpec((B, D2), lambda k: (0, 0)),
        out_shape=jax.ShapeDtypeStruct((B, D2), jnp.float32),
    )(feat, Wd, bd)


def kernel(x, W_enc, b_enc, W_dec, b_dec):
    xf = x.reshape(B, D2)
    be = b_enc.reshape(1, F)
    Wd = W_dec.reshape(F, D2)
    bd = b_dec.reshape(1, D2)

    pre = _encode(xf, W_enc, be)
    features = _topk_mask(pre)
    recon = _decode(features, Wd, bd).reshape(B, 2, D2 // 2)
    return recon, features
